# baseline (speedup 1.0000x reference)
import jax
import jax.numpy as jnp
from jax.experimental import pallas as pl
from jax.experimental.pallas import tpu as pltpu


def _pick_block_rows(n: int) -> int:
    best = 8
    for bm in range(8, min(n, 448) + 1, 8):
        if n % bm == 0:
            best = bm
    return best


def _linear_kernel(x_ref, w_ref, b_ref, h_ref):
    h = jax.lax.dot_general(
        x_ref[...], w_ref[...],
        (((1,), (1,)), ((), ())),
        preferred_element_type=jnp.float32,
    ) + b_ref[...]
    h_ref[...] = h.astype(jnp.bfloat16)


def _spmm_kernel(adj_ref, h_ref, out_ref):
    out_ref[...] = jnp.dot(
        adj_ref[...].astype(jnp.bfloat16), h_ref[...],
        preferred_element_type=jnp.float32,
    )


def kernel(x, adj, W, b):
    n, d_in = x.shape
    d_out = W.shape[0]
    h = pl.pallas_call(
        _linear_kernel,
        out_shape=jax.ShapeDtypeStruct((n, d_out), jnp.bfloat16),
    )(x, W, b.reshape(1, -1))

    bm = _pick_block_rows(n)
    grid = (n // bm,)
    return pl.pallas_call(
        _spmm_kernel,
        grid=grid,
        in_specs=[
            pl.BlockSpec((bm, n), lambda i: (i, 0)),
            pl.BlockSpec((n, d_out), lambda i: (0, 0)),
        ],
        out_specs=pl.BlockSpec((bm, d_out), lambda i: (i, 0)),
        out_shape=jax.ShapeDtypeStruct((n, d_out), jnp.float32),
        compiler_params=pltpu.CompilerParams(
            dimension_semantics=("parallel",),
            vmem_limit_bytes=100 * 1024 * 1024,
        ),
    )(adj, h)


# speedup vs baseline: 1.0131x; 1.0131x over previous
import jax
import jax.numpy as jnp
from jax.experimental import pallas as pl
from jax.experimental.pallas import tpu as pltpu


def _pick_block_rows(nh: int) -> int:
    best = 8
    for bm in range(8, min(nh, 256) + 1, 8):
        if nh % bm == 0:
            best = bm
    return best


def _gc_kernel(x_ref, w_ref, b_ref, adjt_ref, adjb_ref, out_ref, h_ref):
    @pl.when(pl.program_id(0) == 0)
    def _compute_h():
        h = jax.lax.dot_general(
            x_ref[...], w_ref[...],
            (((1,), (1,)), ((), ())),
            preferred_element_type=jnp.float32,
        ) + b_ref[...]
        h_ref[...] = h.astype(jnp.bfloat16)

    hb = h_ref[...]
    out_ref[0] = jnp.dot(
        adjt_ref[...].astype(jnp.bfloat16), hb,
        preferred_element_type=jnp.float32,
    )
    out_ref[1] = jnp.dot(
        adjb_ref[...].astype(jnp.bfloat16), hb,
        preferred_element_type=jnp.float32,
    )


def kernel(x, adj, W, b):
    n, d_in = x.shape
    d_out = W.shape[0]
    nh = n // 2
    bm = _pick_block_rows(nh)
    half_blocks = nh // bm
    grid = (half_blocks,)
    out3 = pl.pallas_call(
        _gc_kernel,
        grid=grid,
        in_specs=[
            pl.BlockSpec((n, d_in), lambda i: (0, 0)),
            pl.BlockSpec((d_out, d_in), lambda i: (0, 0)),
            pl.BlockSpec((1, d_out), lambda i: (0, 0)),
            pl.BlockSpec((bm, n), lambda i: (i, 0)),
            pl.BlockSpec((bm, n), lambda i, hb=half_blocks: (i + hb, 0)),
        ],
        out_specs=pl.BlockSpec((2, bm, d_out), lambda i: (0, i, 0)),
        out_shape=jax.ShapeDtypeStruct((2, nh, d_out), jnp.float32),
        scratch_shapes=[pltpu.VMEM((n, d_out), jnp.bfloat16)],
        compiler_params=pltpu.CompilerParams(
            dimension_semantics=("arbitrary",),
            vmem_limit_bytes=100 * 1024 * 1024,
        ),
    )(x, W, b.reshape(1, -1), adj, adj)
    return out3.reshape(n, d_out)


# speedup vs baseline: 1.0318x; 1.0184x over previous
import jax
import jax.numpy as jnp
from jax.experimental import pallas as pl
from jax.experimental.pallas import tpu as pltpu


def _pick_block_rows(n: int) -> int:
    best = 8
    for bm in range(8, min(n, 448) + 1, 8):
        if n % bm == 0:
            best = bm
    return best


def _gc_kernel(x_ref, w_ref, b_ref, adj_ref, out_ref, h_ref):
    @pl.when(pl.program_id(0) == 0)
    def _compute_h():
        h = jax.lax.dot_general(
            x_ref[...], w_ref[...],
            (((1,), (1,)), ((), ())),
            preferred_element_type=jnp.float32,
        ) + b_ref[...]
        h_ref[...] = h.astype(jnp.bfloat16)

    out_ref[...] = jnp.dot(
        adj_ref[...].astype(jnp.bfloat16), h_ref[...],
        preferred_element_type=jnp.float32,
    )


def kernel(x, adj, W, b):
    n, d_in = x.shape
    d_out = W.shape[0]
    bm = _pick_block_rows(n)
    grid = (n // bm,)
    return pl.pallas_call(
        _gc_kernel,
        grid=grid,
        in_specs=[
            pl.BlockSpec((n, d_in), lambda i: (0, 0)),
            pl.BlockSpec((d_out, d_in), lambda i: (0, 0)),
            pl.BlockSpec((1, d_out), lambda i: (0, 0)),
            pl.BlockSpec((bm, n), lambda i: (i, 0)),
        ],
        out_specs=pl.BlockSpec((bm, d_out), lambda i: (i, 0)),
        out_shape=jax.ShapeDtypeStruct((n, d_out), jnp.float32),
        scratch_shapes=[pltpu.VMEM((n, d_out), jnp.bfloat16)],
        compiler_params=pltpu.CompilerParams(
            dimension_semantics=("arbitrary",),
            vmem_limit_bytes=100 * 1024 * 1024,
        ),
    )(x, W, b.reshape(1, -1), adj)


# speedup vs baseline: 1.0328x; 1.0010x over previous
import jax
import jax.numpy as jnp
from jax.experimental import pallas as pl
from jax.experimental.pallas import tpu as pltpu


def _pick_block_rows(n: int) -> int:
    best = 8
    for bm in range(8, min(n, 448) + 1, 8):
        if n % bm == 0:
            best = bm
    return best


def _gc_kernel(x_ref, w_ref, b_ref, adj_ref, out_ref, h_ref):
    @pl.when(pl.program_id(0) == 0)
    def _compute_h():
        h = jax.lax.dot_general(
            x_ref[...], w_ref[...],
            (((1,), (1,)), ((), ())),
            preferred_element_type=jnp.float32,
        ) + b_ref[...]
        h_ref[...] = h

    out_ref[...] = jnp.dot(
        adj_ref[...], h_ref[...],
        preferred_element_type=jnp.float32,
        precision=jax.lax.Precision.DEFAULT,
    )


def kernel(x, adj, W, b):
    n, d_in = x.shape
    d_out = W.shape[0]
    bm = _pick_block_rows(n)
    grid = (n // bm,)
    return pl.pallas_call(
        _gc_kernel,
        grid=grid,
        in_specs=[
            pl.BlockSpec((n, d_in), lambda i: (0, 0)),
            pl.BlockSpec((d_out, d_in), lambda i: (0, 0)),
            pl.BlockSpec((1, d_out), lambda i: (0, 0)),
            pl.BlockSpec((bm, n), lambda i: (i, 0)),
        ],
        out_specs=pl.BlockSpec((bm, d_out), lambda i: (i, 0)),
        out_shape=jax.ShapeDtypeStruct((n, d_out), jnp.float32),
        scratch_shapes=[pltpu.VMEM((n, d_out), jnp.float32)],
        compiler_params=pltpu.CompilerParams(
            dimension_semantics=("arbitrary",),
            vmem_limit_bytes=100 * 1024 * 1024,
        ),
    )(x, W, b.reshape(1, -1), adj)
